# baseline (speedup 1.0000x reference)
import functools

import jax
import jax.numpy as jnp
from jax import lax
from jax.experimental import pallas as pl
from jax.experimental.pallas import tpu as pltpu
from jax.experimental.pallas import tpu_sc as plsc

N = 10000
E = 320000
D = 128
H = 128

NC = 2
NS = 16
EPW = E // (NC * NS)
K = 40
CHUNKS = EPW // K
RPS = N // NS

W_INFLIGHT = 8


def _deg_body(dst_hbm, ones_hbm, zeros_hbm, out_hbm, didx_all, ones_v, acc, ssem):
    c = lax.axis_index("c")
    s = lax.axis_index("s")
    w = c * NS + s
    pltpu.sync_copy(zeros_hbm, acc.at[pl.ds(s * RPS, RPS)])
    pltpu.sync_copy(dst_hbm.at[w], didx_all)
    pltpu.sync_copy(ones_hbm, ones_v)
    plsc.subcore_barrier()

    for j in range(W_INFLIGHT):
        pltpu.async_copy(ones_v, acc.at[didx_all.at[j]], ssem, add=True)

    def chunk(j, carry):
        pltpu.make_async_copy(ones_v, acc.at[didx_all.at[0]], ssem).wait()
        pltpu.async_copy(ones_v, acc.at[didx_all.at[j + W_INFLIGHT]], ssem,
                         add=True)
        return carry

    lax.fori_loop(0, CHUNKS - W_INFLIGHT, chunk, 0)
    for _ in range(W_INFLIGHT):
        pltpu.make_async_copy(ones_v, acc.at[didx_all.at[0]], ssem).wait()
    plsc.subcore_barrier()
    pltpu.sync_copy(acc.at[pl.ds(s * RPS, RPS)], out_hbm.at[c, s])


NB = 5
G = CHUNKS // NB


def _edge_body(g_hbm, src_hbm, dst_hbm, zeros_hbm, out_hbm,
               sidx_all, didx_all, rows, acc, gsem):
    c = lax.axis_index("c")
    s = lax.axis_index("s")
    w = c * NS + s
    pltpu.sync_copy(zeros_hbm, acc.at[pl.ds(s * RPS, RPS)])
    pltpu.sync_copy(src_hbm.at[w], sidx_all)
    pltpu.sync_copy(dst_hbm.at[w], didx_all)
    plsc.subcore_barrier()

    for b in range(NB):
        pltpu.async_copy(g_hbm.at[sidx_all.at[b]], rows.at[b], gsem.at[b])

    def group(gi, carry):
        for b in range(NB):
            j = gi * NB + b
            pltpu.make_async_copy(g_hbm.at[sidx_all.at[0]], rows.at[b],
                                  gsem.at[b]).wait()
            pltpu.sync_copy(rows.at[b], acc.at[didx_all.at[j]], add=True)
            pltpu.async_copy(g_hbm.at[sidx_all.at[j + NB]], rows.at[b],
                             gsem.at[b])
        return carry

    lax.fori_loop(0, G - 1, group, 0)
    for b in range(NB):
        j = (G - 1) * NB + b
        pltpu.make_async_copy(g_hbm.at[sidx_all.at[0]], rows.at[b],
                              gsem.at[b]).wait()
        pltpu.sync_copy(rows.at[b], acc.at[didx_all.at[j]], add=True)
    plsc.subcore_barrier()
    pltpu.sync_copy(acc.at[pl.ds(s * RPS, RPS)], out_hbm.at[c, s])


@functools.cache
def _sc_kernels():
    mesh = plsc.VectorSubcoreMesh(
        core_axis_name="c", subcore_axis_name="s",
        num_cores=NC, num_subcores=NS,
    )
    params = pltpu.CompilerParams(use_tc_tiling_on_sc=False)
    deg_kernel = pl.kernel(
        _deg_body,
        out_type=jax.ShapeDtypeStruct((NC, NS, RPS, 16), jnp.float32),
        mesh=mesh,
        compiler_params=params,
        scratch_types=[
            pltpu.VMEM((CHUNKS, K), jnp.int32),
            pltpu.VMEM((K, 16), jnp.float32),
            pltpu.VMEM_SHARED((N, 16), jnp.float32),
            pltpu.SemaphoreType.DMA,
        ],
    )
    edge_kernel = pl.kernel(
        _edge_body,
        out_type=jax.ShapeDtypeStruct((NC, NS, RPS, D), jnp.float32),
        mesh=mesh,
        compiler_params=params,
        scratch_types=[
            pltpu.VMEM((CHUNKS, K), jnp.int32),
            pltpu.VMEM((CHUNKS, K), jnp.int32),
            pltpu.VMEM((NB, K, D), jnp.float32),
            pltpu.VMEM_SHARED((N, D), jnp.float32),
            pltpu.SemaphoreType.DMA((NB,)),
        ],
    )
    return deg_kernel, edge_kernel


BLK = 1000


def _mm_body(z_ref, wg_ref, h_ref):
    h_ref[...] = jnp.dot(z_ref[...], wg_ref[...],
                         preferred_element_type=jnp.float32)


def _tc_mm(z, wg):
    return pl.pallas_call(
        _mm_body,
        grid=(N // BLK,),
        in_specs=[
            pl.BlockSpec((BLK, D), lambda i: (i, 0)),
            pl.BlockSpec((D, D), lambda i: (0, 0)),
        ],
        out_specs=pl.BlockSpec((BLK, D), lambda i: (i, 0)),
        out_shape=jax.ShapeDtypeStruct((N, D), jnp.float32),
    )(z, wg)


def _scale_body(h_ref, degp_ref, g_ref):
    deg = degp_ref[0, :, 0] + degp_ref[1, :, 0]
    dinv = lax.rsqrt(jnp.maximum(deg, 1.0))
    g_ref[...] = h_ref[...] * dinv[:, None]


def _tc_scale(h, degp):
    return pl.pallas_call(
        _scale_body,
        grid=(N // BLK,),
        in_specs=[
            pl.BlockSpec((BLK, D), lambda i: (i, 0)),
            pl.BlockSpec((NC, BLK, 16), lambda i: (0, i, 0)),
        ],
        out_specs=pl.BlockSpec((BLK, D), lambda i: (i, 0)),
        out_shape=jax.ShapeDtypeStruct((N, D), jnp.float32),
    )(h, degp)


def _mlp_body(p_ref, degp_ref, bg_ref, w1_ref, b1_ref, w2_ref, b2_ref,
              w3_ref, b3_ref, o_ref):
    deg = degp_ref[0, :, 0] + degp_ref[1, :, 0]
    dinv = lax.rsqrt(jnp.maximum(deg, 1.0))
    agg = (p_ref[0] + p_ref[1]) * dinv[:, None]
    a = jnp.maximum(agg + bg_ref[0][None, :], 0.0)
    h1 = jnp.tanh(jnp.dot(a, w1_ref[...], preferred_element_type=jnp.float32)
                  + b1_ref[0][None, :])
    h2 = jnp.tanh(jnp.dot(h1, w2_ref[...], preferred_element_type=jnp.float32)
                  + b2_ref[0][None, :])
    o_ref[...] = (jnp.dot(h2, w3_ref[...], preferred_element_type=jnp.float32)
                  + b3_ref[0][None, :])


def _tc_mlp(parts, degp, bg, w1, b1, w2, b2, w3, b3):
    mat = lambda: pl.BlockSpec((D, H), lambda i: (0, 0))
    vec = lambda: pl.BlockSpec((1, H), lambda i: (0, 0))
    return pl.pallas_call(
        _mlp_body,
        grid=(N // BLK,),
        in_specs=[
            pl.BlockSpec((NC, BLK, D), lambda i: (0, i, 0)),
            pl.BlockSpec((NC, BLK, 16), lambda i: (0, i, 0)),
            vec(), mat(), vec(), mat(), vec(), mat(), vec(),
        ],
        out_specs=pl.BlockSpec((BLK, D), lambda i: (i, 0)),
        out_shape=jax.ShapeDtypeStruct((N, D), jnp.float32),
    )(parts, degp, bg.reshape(1, D), w1, b1.reshape(1, H), w2,
      b2.reshape(1, H), w3, b3.reshape(1, D))


def kernel(t, z, edge_index, Wg, bg, W1, b1, W2, b2, W3, b3):
    src = edge_index[0].reshape(NC * NS, CHUNKS, K)
    dst = edge_index[1].reshape(NC * NS, CHUNKS, K)
    ones16 = jnp.ones((K, 16), jnp.float32)
    zeros16 = jnp.zeros((RPS, 16), jnp.float32)
    zerosD = jnp.zeros((RPS, D), jnp.float32)

    deg_kernel, edge_kernel = _sc_kernels()
    h = _tc_mm(z, Wg)
    degp = deg_kernel(dst, ones16, zeros16).reshape(NC, N, 16)
    g = _tc_scale(h, degp)
    parts = edge_kernel(g, src, dst, zerosD).reshape(NC, N, D)
    return _tc_mlp(parts, degp, bg, W1, b1, W2, b2, W3, b3)


# speedup vs baseline: 1.0304x; 1.0304x over previous
import functools

import jax
import jax.numpy as jnp
from jax import lax
from jax.experimental import pallas as pl
from jax.experimental.pallas import tpu as pltpu
from jax.experimental.pallas import tpu_sc as plsc

N = 10000
E = 320000
D = 128
H = 128

NC = 2
NS = 16
EPW = E // (NC * NS)
K = 40
CHUNKS = EPW // K
RPS = N // NS
DW = 8

W_INFLIGHT = 8


def _deg_body(dst_hbm, ones_hbm, zeros_hbm, out_hbm, didx_all, ones_v, acc, ssem):
    c = lax.axis_index("c")
    s = lax.axis_index("s")
    w = c * NS + s
    pltpu.sync_copy(zeros_hbm, acc.at[pl.ds(s * RPS, RPS)])
    pltpu.sync_copy(dst_hbm.at[w], didx_all)
    pltpu.sync_copy(ones_hbm, ones_v)
    plsc.subcore_barrier()

    for j in range(W_INFLIGHT):
        pltpu.async_copy(ones_v, acc.at[didx_all.at[j]], ssem, add=True)

    def chunk(j, carry):
        pltpu.make_async_copy(ones_v, acc.at[didx_all.at[0]], ssem).wait()
        pltpu.async_copy(ones_v, acc.at[didx_all.at[j + W_INFLIGHT]], ssem,
                         add=True)
        return carry

    lax.fori_loop(0, CHUNKS - W_INFLIGHT, chunk, 0)
    for _ in range(W_INFLIGHT):
        pltpu.make_async_copy(ones_v, acc.at[didx_all.at[0]], ssem).wait()
    plsc.subcore_barrier()
    pltpu.sync_copy(acc.at[pl.ds(s * RPS, RPS)], out_hbm.at[c, s])


NB = 5
G = CHUNKS // NB


def _edge_body(g_hbm, src_hbm, dst_hbm, zeros_hbm, out_hbm,
               sidx_all, didx_all, rows, acc, gsem):
    c = lax.axis_index("c")
    s = lax.axis_index("s")
    w = c * NS + s
    pltpu.sync_copy(zeros_hbm, acc.at[pl.ds(s * RPS, RPS)])
    pltpu.sync_copy(src_hbm.at[w], sidx_all)
    pltpu.sync_copy(dst_hbm.at[w], didx_all)
    plsc.subcore_barrier()

    for b in range(NB):
        pltpu.async_copy(g_hbm.at[sidx_all.at[b]], rows.at[b], gsem.at[b])

    def group(gi, carry):
        for b in range(NB):
            j = gi * NB + b
            pltpu.make_async_copy(g_hbm.at[sidx_all.at[0]], rows.at[b],
                                  gsem.at[b]).wait()
            pltpu.sync_copy(rows.at[b], acc.at[didx_all.at[j]], add=True)
            pltpu.async_copy(g_hbm.at[sidx_all.at[j + NB]], rows.at[b],
                             gsem.at[b])
        return carry

    lax.fori_loop(0, G - 1, group, 0)
    for b in range(NB):
        j = (G - 1) * NB + b
        pltpu.make_async_copy(g_hbm.at[sidx_all.at[0]], rows.at[b],
                              gsem.at[b]).wait()
        pltpu.sync_copy(rows.at[b], acc.at[didx_all.at[j]], add=True)
    plsc.subcore_barrier()
    pltpu.sync_copy(acc.at[pl.ds(s * RPS, RPS)], out_hbm.at[c, s])


@functools.cache
def _sc_kernels():
    mesh = plsc.VectorSubcoreMesh(
        core_axis_name="c", subcore_axis_name="s",
        num_cores=NC, num_subcores=NS,
    )
    params = pltpu.CompilerParams(use_tc_tiling_on_sc=False)
    deg_kernel = pl.kernel(
        _deg_body,
        out_type=jax.ShapeDtypeStruct((NC, NS, RPS, DW), jnp.float32),
        mesh=mesh,
        compiler_params=params,
        scratch_types=[
            pltpu.VMEM((CHUNKS, K), jnp.int32),
            pltpu.VMEM((K, DW), jnp.float32),
            pltpu.VMEM_SHARED((N, DW), jnp.float32),
            pltpu.SemaphoreType.DMA,
        ],
    )
    edge_kernel = pl.kernel(
        _edge_body,
        out_type=jax.ShapeDtypeStruct((NC, NS, RPS, D), jnp.float32),
        mesh=mesh,
        compiler_params=params,
        scratch_types=[
            pltpu.VMEM((CHUNKS, K), jnp.int32),
            pltpu.VMEM((CHUNKS, K), jnp.int32),
            pltpu.VMEM((NB, K, D), jnp.float32),
            pltpu.VMEM_SHARED((N, D), jnp.float32),
            pltpu.SemaphoreType.DMA((NB,)),
        ],
    )
    return deg_kernel, edge_kernel


BLK = 1000


def _scale_body(z_ref, wg_ref, degp_ref, g_ref):
    deg = degp_ref[0, :, 0] + degp_ref[1, :, 0]
    dinv = lax.rsqrt(jnp.maximum(deg, 1.0))
    h = jnp.dot(z_ref[...], wg_ref[...], preferred_element_type=jnp.float32)
    g_ref[...] = h * dinv[:, None]


def _tc_scale(z, wg, degp):
    return pl.pallas_call(
        _scale_body,
        grid=(N // BLK,),
        in_specs=[
            pl.BlockSpec((BLK, D), lambda i: (i, 0)),
            pl.BlockSpec((D, D), lambda i: (0, 0)),
            pl.BlockSpec((NC, BLK, DW), lambda i: (0, i, 0)),
        ],
        out_specs=pl.BlockSpec((BLK, D), lambda i: (i, 0)),
        out_shape=jax.ShapeDtypeStruct((N, D), jnp.float32),
    )(z, wg, degp)


def _mlp_body(p_ref, degp_ref, bg_ref, w1_ref, b1_ref, w2_ref, b2_ref,
              w3_ref, b3_ref, o_ref):
    deg = degp_ref[0, :, 0] + degp_ref[1, :, 0]
    dinv = lax.rsqrt(jnp.maximum(deg, 1.0))
    agg = (p_ref[0] + p_ref[1]) * dinv[:, None]
    a = jnp.maximum(agg + bg_ref[0][None, :], 0.0)
    h1 = jnp.tanh(jnp.dot(a, w1_ref[...], preferred_element_type=jnp.float32)
                  + b1_ref[0][None, :])
    h2 = jnp.tanh(jnp.dot(h1, w2_ref[...], preferred_element_type=jnp.float32)
                  + b2_ref[0][None, :])
    o_ref[...] = (jnp.dot(h2, w3_ref[...], preferred_element_type=jnp.float32)
                  + b3_ref[0][None, :])


def _tc_mlp(parts, degp, bg, w1, b1, w2, b2, w3, b3):
    mat = lambda: pl.BlockSpec((D, H), lambda i: (0, 0))
    vec = lambda: pl.BlockSpec((1, H), lambda i: (0, 0))
    return pl.pallas_call(
        _mlp_body,
        grid=(N // BLK,),
        in_specs=[
            pl.BlockSpec((NC, BLK, D), lambda i: (0, i, 0)),
            pl.BlockSpec((NC, BLK, DW), lambda i: (0, i, 0)),
            vec(), mat(), vec(), mat(), vec(), mat(), vec(),
        ],
        out_specs=pl.BlockSpec((BLK, D), lambda i: (i, 0)),
        out_shape=jax.ShapeDtypeStruct((N, D), jnp.float32),
    )(parts, degp, bg.reshape(1, D), w1, b1.reshape(1, H), w2,
      b2.reshape(1, H), w3, b3.reshape(1, D))


def kernel(t, z, edge_index, Wg, bg, W1, b1, W2, b2, W3, b3):
    src = edge_index[0].reshape(NC * NS, CHUNKS, K)
    dst = edge_index[1].reshape(NC * NS, CHUNKS, K)
    ones16 = jnp.ones((K, DW), jnp.float32)
    zeros16 = jnp.zeros((RPS, DW), jnp.float32)
    zerosD = jnp.zeros((RPS, D), jnp.float32)

    deg_kernel, edge_kernel = _sc_kernels()
    degp = deg_kernel(dst, ones16, zeros16).reshape(NC, N, DW)
    g = _tc_scale(z, Wg, degp)
    parts = edge_kernel(g, src, dst, zerosD).reshape(NC, N, D)
    return _tc_mlp(parts, degp, bg, W1, b1, W2, b2, W3, b3)


# speedup vs baseline: 1.1451x; 1.1113x over previous
import functools

import jax
import jax.numpy as jnp
from jax import lax
from jax.experimental import pallas as pl
from jax.experimental.pallas import tpu as pltpu
from jax.experimental.pallas import tpu_sc as plsc

N = 10000
E = 320000
D = 128
H = 128

NC = 2
NS = 16
EPW = E // (NC * NS)
K = 40
CHUNKS = EPW // K
RPS = N // NS
DW = 8

W_INFLIGHT = 8


def _deg_body(ei_hbm, ones_hbm, zeros_hbm, out_hbm, didx_all, ones_v, acc, ssem):
    c = lax.axis_index("c")
    s = lax.axis_index("s")
    w = c * NS + s
    pltpu.sync_copy(zeros_hbm, acc.at[pl.ds(s * RPS, RPS)])
    pltpu.sync_copy(ei_hbm.at[1, w], didx_all)
    pltpu.sync_copy(ones_hbm, ones_v)
    plsc.subcore_barrier()

    for j in range(W_INFLIGHT):
        pltpu.async_copy(ones_v, acc.at[didx_all.at[j]], ssem, add=True)

    def chunk(j, carry):
        pltpu.make_async_copy(ones_v, acc.at[didx_all.at[0]], ssem).wait()
        pltpu.async_copy(ones_v, acc.at[didx_all.at[j + W_INFLIGHT]], ssem,
                         add=True)
        return carry

    lax.fori_loop(0, CHUNKS - W_INFLIGHT, chunk, 0)
    for _ in range(W_INFLIGHT):
        pltpu.make_async_copy(ones_v, acc.at[didx_all.at[0]], ssem).wait()
    plsc.subcore_barrier()
    pltpu.sync_copy(acc.at[pl.ds(s * RPS, RPS)], out_hbm.at[c, s])


NB = 5
G = CHUNKS // NB


def _edge_body(g_hbm, ei_hbm, zeros_hbm, out_hbm,
               sidx_all, didx_all, rows, acc, gsem):
    c = lax.axis_index("c")
    s = lax.axis_index("s")
    w = c * NS + s
    pltpu.sync_copy(zeros_hbm, acc.at[pl.ds(s * RPS, RPS)])
    pltpu.sync_copy(ei_hbm.at[0, w], sidx_all)
    pltpu.sync_copy(ei_hbm.at[1, w], didx_all)
    plsc.subcore_barrier()

    for b in range(NB):
        pltpu.async_copy(g_hbm.at[sidx_all.at[b]], rows.at[b], gsem.at[b])

    def group(gi, carry):
        for b in range(NB):
            j = gi * NB + b
            pltpu.make_async_copy(g_hbm.at[sidx_all.at[0]], rows.at[b],
                                  gsem.at[b]).wait()
            pltpu.sync_copy(rows.at[b], acc.at[didx_all.at[j]], add=True)
            pltpu.async_copy(g_hbm.at[sidx_all.at[j + NB]], rows.at[b],
                             gsem.at[b])
        return carry

    lax.fori_loop(0, G - 1, group, 0)
    for b in range(NB):
        j = (G - 1) * NB + b
        pltpu.make_async_copy(g_hbm.at[sidx_all.at[0]], rows.at[b],
                              gsem.at[b]).wait()
        pltpu.sync_copy(rows.at[b], acc.at[didx_all.at[j]], add=True)
    plsc.subcore_barrier()
    pltpu.sync_copy(acc.at[pl.ds(s * RPS, RPS)], out_hbm.at[c, s])


@functools.cache
def _sc_kernels():
    mesh = plsc.VectorSubcoreMesh(
        core_axis_name="c", subcore_axis_name="s",
        num_cores=NC, num_subcores=NS,
    )
    params = pltpu.CompilerParams(use_tc_tiling_on_sc=False)
    deg_kernel = pl.kernel(
        _deg_body,
        out_type=jax.ShapeDtypeStruct((NC, NS, RPS, DW), jnp.float32),
        mesh=mesh,
        compiler_params=params,
        scratch_types=[
            pltpu.VMEM((CHUNKS, K), jnp.int32),
            pltpu.VMEM((K, DW), jnp.float32),
            pltpu.VMEM_SHARED((N, DW), jnp.float32),
            pltpu.SemaphoreType.DMA,
        ],
    )
    edge_kernel = pl.kernel(
        _edge_body,
        out_type=jax.ShapeDtypeStruct((NC, NS, RPS, D), jnp.float32),
        mesh=mesh,
        compiler_params=params,
        scratch_types=[
            pltpu.VMEM((CHUNKS, K), jnp.int32),
            pltpu.VMEM((CHUNKS, K), jnp.int32),
            pltpu.VMEM((NB, K, D), jnp.float32),
            pltpu.VMEM_SHARED((N, D), jnp.float32),
            pltpu.SemaphoreType.DMA((NB,)),
        ],
    )
    return deg_kernel, edge_kernel


BLK = 2000


def _scale_body(z_ref, wg_ref, degp_ref, g_ref):
    deg = degp_ref[0, :, 0] + degp_ref[1, :, 0]
    dinv = lax.rsqrt(jnp.maximum(deg, 1.0))
    h = jnp.dot(z_ref[...], wg_ref[...], preferred_element_type=jnp.float32)
    g_ref[...] = h * dinv[:, None]


def _tc_scale(z, wg, degp):
    return pl.pallas_call(
        _scale_body,
        grid=(N // BLK,),
        in_specs=[
            pl.BlockSpec((BLK, D), lambda i: (i, 0)),
            pl.BlockSpec((D, D), lambda i: (0, 0)),
            pl.BlockSpec((NC, BLK, DW), lambda i: (0, i, 0)),
        ],
        out_specs=pl.BlockSpec((BLK, D), lambda i: (i, 0)),
        out_shape=jax.ShapeDtypeStruct((N, D), jnp.float32),
    )(z, wg, degp)


def _mlp_body(p_ref, degp_ref, bg_ref, w1_ref, b1_ref, w2_ref, b2_ref,
              w3_ref, b3_ref, o_ref):
    deg = degp_ref[0, :, 0] + degp_ref[1, :, 0]
    dinv = lax.rsqrt(jnp.maximum(deg, 1.0))
    agg = (p_ref[0] + p_ref[1]) * dinv[:, None]
    a = jnp.maximum(agg + bg_ref[0][None, :], 0.0)
    h1 = jnp.tanh(jnp.dot(a, w1_ref[...], preferred_element_type=jnp.float32)
                  + b1_ref[0][None, :])
    h2 = jnp.tanh(jnp.dot(h1, w2_ref[...], preferred_element_type=jnp.float32)
                  + b2_ref[0][None, :])
    o_ref[...] = (jnp.dot(h2, w3_ref[...], preferred_element_type=jnp.float32)
                  + b3_ref[0][None, :])


def _tc_mlp(parts, degp, bg, w1, b1, w2, b2, w3, b3):
    mat = lambda: pl.BlockSpec((D, H), lambda i: (0, 0))
    vec = lambda: pl.BlockSpec((1, H), lambda i: (0, 0))
    return pl.pallas_call(
        _mlp_body,
        grid=(N // BLK,),
        in_specs=[
            pl.BlockSpec((NC, BLK, D), lambda i: (0, i, 0)),
            pl.BlockSpec((NC, BLK, DW), lambda i: (0, i, 0)),
            vec(), mat(), vec(), mat(), vec(), mat(), vec(),
        ],
        out_specs=pl.BlockSpec((BLK, D), lambda i: (i, 0)),
        out_shape=jax.ShapeDtypeStruct((N, D), jnp.float32),
    )(parts, degp, bg.reshape(1, D), w1, b1.reshape(1, H), w2,
      b2.reshape(1, H), w3, b3.reshape(1, D))


def kernel(t, z, edge_index, Wg, bg, W1, b1, W2, b2, W3, b3):
    ei4 = edge_index.reshape(2, NC * NS, CHUNKS, K)
    ones16 = jnp.ones((K, DW), jnp.float32)
    zeros16 = jnp.zeros((RPS, DW), jnp.float32)
    zerosD = jnp.zeros((RPS, D), jnp.float32)

    deg_kernel, edge_kernel = _sc_kernels()
    degp = deg_kernel(ei4, ones16, zeros16).reshape(NC, N, DW)
    g = _tc_scale(z, Wg, degp)
    parts = edge_kernel(g, ei4, zerosD).reshape(NC, N, D)
    return _tc_mlp(parts, degp, bg, W1, b1, W2, b2, W3, b3)


# speedup vs baseline: 1.1590x; 1.0122x over previous
import functools

import jax
import jax.numpy as jnp
from jax import lax
from jax.experimental import pallas as pl
from jax.experimental.pallas import tpu as pltpu
from jax.experimental.pallas import tpu_sc as plsc

N = 10000
E = 320000
D = 128
H = 128

NC = 2
NS = 16
EPW = E // (NC * NS)
K = 40
CHUNKS = EPW // K
RPS = N // NS
DW = 8

W_INFLIGHT = 16


def _deg_body(ei_hbm, ones_hbm, zeros_hbm, out_hbm, didx_all, ones_v, acc, ssem):
    c = lax.axis_index("c")
    s = lax.axis_index("s")
    w = c * NS + s
    pltpu.sync_copy(zeros_hbm, acc.at[pl.ds(s * RPS, RPS)])
    pltpu.sync_copy(ei_hbm.at[1, w], didx_all)
    pltpu.sync_copy(ones_hbm, ones_v)
    plsc.subcore_barrier()

    for j in range(W_INFLIGHT):
        pltpu.async_copy(ones_v, acc.at[didx_all.at[j]], ssem, add=True)

    def chunk(j, carry):
        pltpu.make_async_copy(ones_v, acc.at[didx_all.at[0]], ssem).wait()
        pltpu.async_copy(ones_v, acc.at[didx_all.at[j + W_INFLIGHT]], ssem,
                         add=True)
        return carry

    lax.fori_loop(0, CHUNKS - W_INFLIGHT, chunk, 0)
    for _ in range(W_INFLIGHT):
        pltpu.make_async_copy(ones_v, acc.at[didx_all.at[0]], ssem).wait()
    plsc.subcore_barrier()
    pltpu.sync_copy(acc.at[pl.ds(s * RPS, RPS)], out_hbm.at[c, s])


NB = 5
G = CHUNKS // NB


def _edge_body(g_hbm, ei_hbm, zeros_hbm, out_hbm,
               sidx_all, didx_all, rows, acc, gsem):
    c = lax.axis_index("c")
    s = lax.axis_index("s")
    w = c * NS + s
    pltpu.sync_copy(zeros_hbm, acc.at[pl.ds(s * RPS, RPS)])
    pltpu.sync_copy(ei_hbm.at[0, w], sidx_all)
    pltpu.sync_copy(ei_hbm.at[1, w], didx_all)
    plsc.subcore_barrier()

    for b in range(NB):
        pltpu.async_copy(g_hbm.at[sidx_all.at[b]], rows.at[b], gsem.at[b])

    def group(gi, carry):
        for b in range(NB):
            j = gi * NB + b
            pltpu.make_async_copy(g_hbm.at[sidx_all.at[0]], rows.at[b],
                                  gsem.at[b]).wait()
            pltpu.sync_copy(rows.at[b], acc.at[didx_all.at[j]], add=True)
            pltpu.async_copy(g_hbm.at[sidx_all.at[j + NB]], rows.at[b],
                             gsem.at[b])
        return carry

    lax.fori_loop(0, G - 1, group, 0)
    for b in range(NB):
        j = (G - 1) * NB + b
        pltpu.make_async_copy(g_hbm.at[sidx_all.at[0]], rows.at[b],
                              gsem.at[b]).wait()
        pltpu.sync_copy(rows.at[b], acc.at[didx_all.at[j]], add=True)
    plsc.subcore_barrier()
    pltpu.sync_copy(acc.at[pl.ds(s * RPS, RPS)], out_hbm.at[c, s])


@functools.cache
def _sc_kernels():
    mesh = plsc.VectorSubcoreMesh(
        core_axis_name="c", subcore_axis_name="s",
        num_cores=NC, num_subcores=NS,
    )
    params = pltpu.CompilerParams(use_tc_tiling_on_sc=False)
    deg_kernel = pl.kernel(
        _deg_body,
        out_type=jax.ShapeDtypeStruct((NC, NS, RPS, DW), jnp.float32),
        mesh=mesh,
        compiler_params=params,
        scratch_types=[
            pltpu.VMEM((CHUNKS, K), jnp.int32),
            pltpu.VMEM((K, DW), jnp.float32),
            pltpu.VMEM_SHARED((N, DW), jnp.float32),
            pltpu.SemaphoreType.DMA,
        ],
    )
    edge_kernel = pl.kernel(
        _edge_body,
        out_type=jax.ShapeDtypeStruct((NC, NS, RPS, D), jnp.float32),
        mesh=mesh,
        compiler_params=params,
        scratch_types=[
            pltpu.VMEM((CHUNKS, K), jnp.int32),
            pltpu.VMEM((CHUNKS, K), jnp.int32),
            pltpu.VMEM((NB, K, D), jnp.float32),
            pltpu.VMEM_SHARED((N, D), jnp.float32),
            pltpu.SemaphoreType.DMA((NB,)),
        ],
    )
    return deg_kernel, edge_kernel


BLK = 5000


def _scale_body(z_ref, wg_ref, degp_ref, g_ref):
    deg = degp_ref[0, :, 0] + degp_ref[1, :, 0]
    dinv = lax.rsqrt(jnp.maximum(deg, 1.0))
    h = jnp.dot(z_ref[...], wg_ref[...], preferred_element_type=jnp.float32,
                precision=lax.Precision.DEFAULT)
    g_ref[...] = h * dinv[:, None]


def _tc_scale(z, wg, degp):
    return pl.pallas_call(
        _scale_body,
        grid=(N // BLK,),
        in_specs=[
            pl.BlockSpec((BLK, D), lambda i: (i, 0)),
            pl.BlockSpec((D, D), lambda i: (0, 0)),
            pl.BlockSpec((NC, BLK, DW), lambda i: (0, i, 0)),
        ],
        out_specs=pl.BlockSpec((BLK, D), lambda i: (i, 0)),
        out_shape=jax.ShapeDtypeStruct((N, D), jnp.float32),
    )(z, wg, degp)


def _mlp_body(p_ref, degp_ref, bg_ref, w1_ref, b1_ref, w2_ref, b2_ref,
              w3_ref, b3_ref, o_ref):
    deg = degp_ref[0, :, 0] + degp_ref[1, :, 0]
    dinv = lax.rsqrt(jnp.maximum(deg, 1.0))
    agg = (p_ref[0] + p_ref[1]) * dinv[:, None]
    a = jnp.maximum(agg + bg_ref[0][None, :], 0.0)
    h1 = jnp.tanh(jnp.dot(a, w1_ref[...], preferred_element_type=jnp.float32,
                          precision=lax.Precision.DEFAULT)
                  + b1_ref[0][None, :])
    h2 = jnp.tanh(jnp.dot(h1, w2_ref[...], preferred_element_type=jnp.float32,
                           precision=lax.Precision.DEFAULT)
                  + b2_ref[0][None, :])
    o_ref[...] = (jnp.dot(h2, w3_ref[...], preferred_element_type=jnp.float32,
                           precision=lax.Precision.DEFAULT)
                  + b3_ref[0][None, :])


def _tc_mlp(parts, degp, bg, w1, b1, w2, b2, w3, b3):
    mat = lambda: pl.BlockSpec((D, H), lambda i: (0, 0))
    vec = lambda: pl.BlockSpec((1, H), lambda i: (0, 0))
    return pl.pallas_call(
        _mlp_body,
        grid=(N // BLK,),
        in_specs=[
            pl.BlockSpec((NC, BLK, D), lambda i: (0, i, 0)),
            pl.BlockSpec((NC, BLK, DW), lambda i: (0, i, 0)),
            vec(), mat(), vec(), mat(), vec(), mat(), vec(),
        ],
        out_specs=pl.BlockSpec((BLK, D), lambda i: (i, 0)),
        out_shape=jax.ShapeDtypeStruct((N, D), jnp.float32),
    )(parts, degp, bg.reshape(1, D), w1, b1.reshape(1, H), w2,
      b2.reshape(1, H), w3, b3.reshape(1, D))


def kernel(t, z, edge_index, Wg, bg, W1, b1, W2, b2, W3, b3):
    ei4 = edge_index.reshape(2, NC * NS, CHUNKS, K)
    ones16 = jnp.ones((K, DW), jnp.float32)
    zeros16 = jnp.zeros((RPS, DW), jnp.float32)
    zerosD = jnp.zeros((RPS, D), jnp.float32)

    deg_kernel, edge_kernel = _sc_kernels()
    degp = deg_kernel(ei4, ones16, zeros16).reshape(NC, N, DW)
    g = _tc_scale(z, Wg, degp)
    parts = edge_kernel(g, ei4, zerosD).reshape(NC, N, D)
    return _tc_mlp(parts, degp, bg, W1, b1, W2, b2, W3, b3)
